# Initial kernel scaffold; baseline (speedup 1.0000x reference)
import jax
import jax.numpy as jnp
from jax.experimental import pallas as pl


def kernel(x1, x2, edges, W1, b1, Wg, bg, W3, b3, W4, b4):
    raise NotImplementedError("write your pallas kernel here")


# speedup vs baseline: 13.5928x; 13.5928x over previous
import functools

import jax
import jax.numpy as jnp
import numpy as np
from jax import lax
from jax.experimental import pallas as pl
from jax.experimental.pallas import tpu as pltpu
from jax.experimental.pallas import tpu_sc as plsc

N = 100000
NP = N + 96
PK = NP // 8
PK_VALID = N // 8
NW = 32
NCORE = 2
NSUB = 16
ZR = NP // NSUB

E = 1600000
NCHT = E // 128
NPASS = 2
CPP = NCHT // NPASS
CPT = CPP // NSUB
XTRA = CPP - CPT * NSUB
HALF = NP // 2
HP = HALF + 128
HPK = HP // 8
HPKV = HALF // 8

BRP = 368
GPK = PK // BRP

BC = 1280
KB = BC // 8
GPRE = -(-N // BC)

_F32 = jnp.float32
_BF16 = jnp.bfloat16


def _make_sc_scatter():
    mesh = plsc.VectorSubcoreMesh(
        core_axis_name="c", subcore_axis_name="s",
        num_cores=NCORE, num_subcores=NSUB)

    @functools.partial(
        pl.kernel,
        out_type=jax.ShapeDtypeStruct((NCORE, HP, 16), _BF16),
        mesh=mesh,
        scratch_types=[
            pltpu.VMEM((CPT + 1, 128), jnp.int32),
            pltpu.VMEM((CPT + 1, 128), jnp.int32),
            pltpu.VMEM((2, 128, 16), _BF16),
            pltpu.VMEM_SHARED((HP, 16), _BF16),
            pltpu.SemaphoreType.DMA,
        ],
        compiler_params=pltpu.CompilerParams(use_tc_tiling_on_sc=False),
    )
    def sc_scatter(y_hbm, edges_hbm, zeros_hbm, out_hbm,
                   ridx, cidx, gbuf, acc, sem):
        c = lax.axis_index("c")
        s = lax.axis_index("s")
        base = c * HALF
        zh = HP // NSUB
        pltpu.sync_copy(zeros_hbm.at[pl.ds(s * zh, zh)],
                        acc.at[pl.ds(s * zh, zh)])
        plsc.subcore_barrier()

        def run_pass(p, carry):
            cbase = p * CPP + s * CPT
            pltpu.sync_copy(edges_hbm.at[pl.ds(cbase, CPT)],
                            ridx.at[pl.ds(0, CPT)])
            pltpu.sync_copy(edges_hbm.at[pl.ds(NCHT + cbase, CPT)],
                            cidx.at[pl.ds(0, CPT)])
            xc = p * CPP + CPP - XTRA + s

            @pl.when(s < XTRA)
            def _():
                pltpu.sync_copy(edges_hbm.at[pl.ds(xc, 1)],
                                ridx.at[pl.ds(CPT, 1)])
                pltpu.sync_copy(edges_hbm.at[pl.ds(NCHT + xc, 1)],
                                cidx.at[pl.ds(CPT, 1)])

            def remap(q, carry2):
                for k in range(8):
                    v = cidx[q, pl.ds(16 * k, 16)] - base
                    bad = jnp.logical_or(v < 0, v >= HALF)
                    cidx[q, pl.ds(16 * k, 16)] = jnp.where(bad, HALF, v)
                return carry2

            lax.fori_loop(0, CPT + 1, remap, 0)

            def body(j, carry2):
                pltpu.async_copy(y_hbm.at[ridx.at[j]], gbuf.at[0], sem).wait()
                pltpu.sync_copy(gbuf.at[0], acc.at[cidx.at[j]], add=True)
                return carry2

            lax.fori_loop(0, CPT, body, 0)

            @pl.when(s < XTRA)
            def _():
                body(CPT, 0)

            return carry

        lax.fori_loop(0, NPASS, run_pass, 0)
        plsc.subcore_barrier()
        pltpu.sync_copy(acc.at[pl.ds(s * zh, zh)],
                        out_hbm.at[c].at[pl.ds(s * zh, zh)])

    return sc_scatter


_SC_CACHE = {}


def _sc_scatter(*args):
    if "k" not in _SC_CACHE:
        _SC_CACHE["k"] = _make_sc_scatter()
    return _SC_CACHE["k"](*args)


def _pre_body(x1t_ref, x2t_ref, wts_ref, A_ref, h0_ref, c2_ref, psum_ref):
    i = pl.program_id(0)
    x1b = x1t_ref[...]
    x2b = x2t_ref[...]
    wts = wts_ref[...]
    W1T = wts[:, 0:15]
    MAT = wts[:, 16:31]
    MBT = wts[:, 32:36]
    b1c = wts[:, 48:49]
    c2c = wts[:, 49:50]

    h0t = jnp.maximum(
        jnp.dot(W1T, x1b, preferred_element_type=_F32) + b1c, 0.0)
    c2t = (jnp.dot(MAT, x1b, preferred_element_type=_F32)
           + jnp.dot(MBT, x2b, preferred_element_type=_F32) + c2c)

    A = A_ref[...]
    h0n = jnp.transpose(h0t)
    c2n = jnp.transpose(c2t)
    h0_ref[...] = jnp.concatenate(
        [jnp.dot(A[r * KB:(r + 1) * KB], h0n, preferred_element_type=_F32)
         for r in range(8)], axis=1)
    c2_ref[...] = jnp.concatenate(
        [jnp.dot(A[r * KB:(r + 1) * KB], c2n, preferred_element_type=_F32)
         for r in range(8)], axis=1)

    lane = i * BC + lax.broadcasted_iota(jnp.int32, (16, BC), 1)
    m = lane < N
    ps1 = jnp.sum(jnp.where(m[:15], x1b, 0.0), axis=1, keepdims=True)
    ps2 = jnp.sum(jnp.where(m[:4], x2b, 0.0), axis=1, keepdims=True)
    col0 = jnp.concatenate([ps1, jnp.zeros((1, 1), _F32)], axis=0)
    col2 = jnp.concatenate([ps2, jnp.zeros((12, 1), _F32)], axis=0)
    ps = jnp.concatenate(
        [col0, jnp.zeros((16, 1), _F32), col2, jnp.zeros((16, 125), _F32)],
        axis=1)

    @pl.when(i == 0)
    def _():
        psum_ref[...] = ps

    @pl.when(i > 0)
    def _():
        psum_ref[...] += ps


def _y0_body(p_ref, h0_ref, c2_ref, spread_ref, WB1_ref, dis_ref, y0_ref):
    cnt = p_ref[0].astype(_F32)
    deg = jnp.dot(cnt, spread_ref[...], preferred_element_type=_F32) + 1.0
    dis = lax.rsqrt(deg)
    dis_ref[...] = dis
    y0 = (jnp.dot(h0_ref[...], WB1_ref[...],
                  preferred_element_type=_F32) + c2_ref[...]) * dis
    y0_ref[...] = y0.astype(_BF16)


def _layer_body(s_ref, y_ref, dis_ref, c2_ref, WB1_ref, bgt_ref, ynew_ref):
    dis = dis_ref[...]
    s = (s_ref[0].astype(_F32) + y_ref[...].astype(_F32))
    h = jnp.maximum(dis * s + bgt_ref[...], 0.0)
    ynew = (jnp.dot(h, WB1_ref[...],
                    preferred_element_type=_F32) + c2_ref[...]) * dis
    ynew_ref[...] = ynew.astype(_BF16)


def _final_body(s_ref, y_ref, dis_ref, bgt_ref, out_ref):
    i = pl.program_id(0)
    s = (s_ref[0].astype(_F32) + y_ref[...].astype(_F32))
    h = jnp.maximum(dis_ref[...] * s + bgt_ref[...], 0.0)
    prid = i * BRP + lax.broadcasted_iota(jnp.int32, (BRP, 128), 0)
    ps = jnp.sum(jnp.where(prid < PK_VALID, h, 0.0), axis=0, keepdims=True)

    @pl.when(i == 0)
    def _():
        out_ref[...] = ps

    @pl.when(i > 0)
    def _():
        out_ref[...] += ps


def _const_spec(shape):
    nd = len(shape)
    return pl.BlockSpec(shape, lambda i: (0,) * nd)


_pre_call = pl.pallas_call(
    _pre_body,
    grid=(GPRE,),
    in_specs=[
        pl.BlockSpec((15, BC), lambda i: (0, i)),
        pl.BlockSpec((4, BC), lambda i: (0, i)),
        _const_spec((16, 128)),
        _const_spec((BC, BC)),
    ],
    out_specs=[
        pl.BlockSpec((KB, 128), lambda i: (i, 0)),
        pl.BlockSpec((KB, 128), lambda i: (i, 0)),
        pl.BlockSpec((16, 128), lambda i: (0, 0)),
    ],
    out_shape=[
        jax.ShapeDtypeStruct((PK, 128), _F32),
        jax.ShapeDtypeStruct((PK, 128), _F32),
        jax.ShapeDtypeStruct((16, 128), _F32),
    ],
)

_y0_call = pl.pallas_call(
    _y0_body,
    grid=(GPK,),
    in_specs=[
        pl.BlockSpec((1, BRP, 128), lambda i: (i // (GPK // 2), i % (GPK // 2), 0)),
        pl.BlockSpec((BRP, 128), lambda i: (i, 0)),
        pl.BlockSpec((BRP, 128), lambda i: (i, 0)),
        _const_spec((128, 128)),
        _const_spec((128, 128)),
    ],
    out_specs=[
        pl.BlockSpec((BRP, 128), lambda i: (i, 0)),
        pl.BlockSpec((BRP, 128), lambda i: (i, 0)),
    ],
    out_shape=[
        jax.ShapeDtypeStruct((PK, 128), _F32),
        jax.ShapeDtypeStruct((PK, 128), _BF16),
    ],
)

_layer_call = pl.pallas_call(
    _layer_body,
    grid=(GPK,),
    in_specs=[
        pl.BlockSpec((1, BRP, 128), lambda i: (i // (GPK // 2), i % (GPK // 2), 0)),
        pl.BlockSpec((BRP, 128), lambda i: (i, 0)),
        pl.BlockSpec((BRP, 128), lambda i: (i, 0)),
        pl.BlockSpec((BRP, 128), lambda i: (i, 0)),
        _const_spec((128, 128)),
        _const_spec((1, 128)),
    ],
    out_specs=pl.BlockSpec((BRP, 128), lambda i: (i, 0)),
    out_shape=jax.ShapeDtypeStruct((PK, 128), _BF16),
)

_final_call = pl.pallas_call(
    _final_body,
    grid=(GPK,),
    in_specs=[
        pl.BlockSpec((1, BRP, 128), lambda i: (i // (GPK // 2), i % (GPK // 2), 0)),
        pl.BlockSpec((BRP, 128), lambda i: (i, 0)),
        pl.BlockSpec((BRP, 128), lambda i: (i, 0)),
        _const_spec((1, 128)),
    ],
    out_specs=pl.BlockSpec((1, 128), lambda i: (0, 0)),
    out_shape=jax.ShapeDtypeStruct((1, 128), _F32),
)

_PERM = np.zeros((BC, BC), np.float32)
for _q in range(BC):
    _PERM[_q, 8 * (_q % KB) + (_q // KB)] = 1.0


def kernel(x1, x2, edges, W1, b1, Wg, bg, W3, b3, W4, b4):
    x1t = jnp.transpose(x1)
    x2t = jnp.tile(jnp.transpose(x2), (1, 20))
    edges_pk = edges.reshape(2 * NCHT, 128)

    W4a = jnp.zeros((16, 32), _F32).at[:15, :19].set(W4[:15])
    W4b16 = jnp.zeros((16, 32), _F32).at[:4, :19].set(W4[15:])
    Wg2 = jnp.zeros((32, 16), _F32).at[:19, :15].set(Wg[15:])
    wts = jnp.zeros((16, 128), _F32)
    wts = wts.at[:15, 0:15].set(W1.T)
    wts = wts.at[:, 16:31].set((W4a @ Wg2).T[:, :15])
    wts = wts.at[:, 32:36].set((W4b16 @ Wg2).T[:, :4])
    wts = wts.at[:15, 48].set(b1)
    wts = wts.at[:15, 49].set(b4 @ Wg[15:])
    A = jnp.asarray(_PERM)

    eye8 = jnp.eye(8, dtype=_F32)
    Wg1 = jnp.zeros((16, 16), _F32).at[:15, :15].set(Wg[:15])
    WB1 = jnp.kron(eye8, Wg1)
    spread = jnp.kron(eye8, jnp.zeros((16, 16), _F32).at[0, :].set(1.0))
    bgt = jnp.tile(jnp.zeros((16,), _F32).at[:15].set(bg), 8)[None, :]

    zeros_np16 = jnp.zeros((HP, 16), _BF16)
    ones_e0 = jnp.zeros((PK, 128), _BF16).at[:, 0::16].set(1.0)

    h0_pk, c2_pk, psums = _pre_call(x1t, x2t, wts, A)

    def step(carry, it):
        y, dis, _, _ = carry
        s = _sc_scatter(y.reshape(NP, 16), edges_pk, zeros_np16)
        s_pk = s.reshape(NCORE, HPK, 128)

        def b0(_):
            d, y0 = _y0_call(s_pk, h0_pk, c2_pk, spread, WB1)
            return d, y0

        def bn(_):
            return dis, _layer_call(s_pk, y, dis, c2_pk, WB1, bgt)

        dis2, ynew = lax.cond(it == 0, b0, bn, None)
        return (ynew, dis2, s_pk, y), None

    carry0 = (ones_e0,
              jnp.zeros((PK, 128), _F32),
              jnp.zeros((NCORE, HPK, 128), _BF16),
              jnp.zeros((PK, 128), _BF16))
    (_, dis, s5_pk, y4), _ = lax.scan(step, carry0, jnp.arange(6))
    ch5 = _final_call(s5_pk, y4, dis, bgt)

    ch5_16 = ch5.reshape(8, 16).sum(axis=0)
    x1sum = psums[:15, 0]
    x2sum = psums[:4, 2]
    bsum = x1sum @ W4[:15] + x2sum @ W4[15:] + N * b4
    val = (ch5_16[:15] @ W3[:15, 0] + bsum @ W3[15:, 0]) / N + b3[0]
    return jnp.tanh(val)


# speedup vs baseline: 45.7380x; 3.3649x over previous
import functools

import jax
import jax.numpy as jnp
import numpy as np
from jax import lax
from jax.experimental import pallas as pl
from jax.experimental.pallas import tpu as pltpu
from jax.experimental.pallas import tpu_sc as plsc

N = 100000
NP = N + 96
PK = NP // 8
PK_VALID = N // 8
NW = 32
NCORE = 2
NSUB = 16
ZR = NP // NSUB

E = 1600000
NCHT = E // 128
CPW = NCHT // NW
XTRA = NCHT - CPW * NW
SP = 98

NBUF = 8
PF = 4
BRP = 368
GPK = PK // BRP

BC = 1280
KB = BC // 8
GPRE = -(-N // BC)

_F32 = jnp.float32
_BF16 = jnp.bfloat16


def _make_sc_scatter():
    mesh = plsc.VectorSubcoreMesh(
        core_axis_name="c", subcore_axis_name="s",
        num_cores=NCORE, num_subcores=NSUB)

    @functools.partial(
        pl.kernel,
        out_type=jax.ShapeDtypeStruct((NCORE, NP, 16), _BF16),
        mesh=mesh,
        scratch_types=[
            pltpu.VMEM((SP, 128), jnp.int32),
            pltpu.VMEM((SP, 128), jnp.int32),
            pltpu.VMEM((NBUF, 128, 16), _BF16),
            pltpu.VMEM_SHARED((NP, 16), _BF16),
            pltpu.SemaphoreType.DMA((NBUF,)),
            pltpu.SemaphoreType.DMA,
        ],
        compiler_params=pltpu.CompilerParams(use_tc_tiling_on_sc=False),
    )
    def sc_scatter(y_hbm, edges_hbm, zeros_hbm, out_hbm,
                   ridx, cidx, gbuf, acc, gsem, ssem):
        c = lax.axis_index("c")
        s = lax.axis_index("s")
        w = c * NSUB + s
        pltpu.sync_copy(zeros_hbm.at[pl.ds(s * ZR, ZR)],
                        acc.at[pl.ds(s * ZR, ZR)])
        plsc.subcore_barrier()

        def gstart(k):
            pltpu.async_copy(y_hbm.at[ridx.at[k]], gbuf.at[k % NBUF],
                             gsem.at[k % NBUF])

        def gwait(j):
            pltpu.make_async_copy(y_hbm.at[ridx.at[j]], gbuf.at[j % NBUF],
                                  gsem.at[j % NBUF]).wait()

        def sdrain():
            pltpu.make_async_copy(y_hbm.at[pl.ds(0, 128)], gbuf.at[0],
                                  ssem).wait()

        for p in range(4):
            lo = p * SP
            cnt = min(SP, CPW - lo)
            pltpu.sync_copy(edges_hbm.at[pl.ds(w * CPW + lo, cnt)],
                            ridx.at[pl.ds(0, cnt)])
            pltpu.sync_copy(edges_hbm.at[pl.ds(NCHT + w * CPW + lo, cnt)],
                            cidx.at[pl.ds(0, cnt)])
            if p == 3:
                @pl.when(w < XTRA)
                def _():
                    pltpu.sync_copy(edges_hbm.at[pl.ds(NW * CPW + w, 1)],
                                    ridx.at[pl.ds(cnt, 1)])
                    pltpu.sync_copy(
                        edges_hbm.at[pl.ds(NCHT + NW * CPW + w, 1)],
                        cidx.at[pl.ds(cnt, 1)])
                nloc = cnt + jnp.where(w < XTRA, 1, 0)
            else:
                nloc = cnt

            for k in range(PF):
                @pl.when(k < nloc)
                def _(k=k):
                    gstart(k)

            def body(j, carry2):
                gwait(j)
                pltpu.async_copy(gbuf.at[j % NBUF], acc.at[cidx.at[j]],
                                 ssem, add=True)

                @pl.when(j >= PF)
                def _():
                    sdrain()

                @pl.when(j + PF < nloc)
                def _():
                    gstart(j + PF)

                return carry2

            lax.fori_loop(0, nloc, body, 0)
            for k in range(PF):
                @pl.when(k < nloc)
                def _():
                    sdrain()

        plsc.subcore_barrier()
        pltpu.sync_copy(acc.at[pl.ds(s * ZR, ZR)],
                        out_hbm.at[c].at[pl.ds(s * ZR, ZR)])

    return sc_scatter


_SC_CACHE = {}


def _sc_scatter(*args):
    if "k" not in _SC_CACHE:
        _SC_CACHE["k"] = _make_sc_scatter()
    return _SC_CACHE["k"](*args)


def _pre_body(x1t_ref, x2t_ref, wts_ref, A_ref, h0_ref, c2_ref, psum_ref):
    i = pl.program_id(0)
    x1b = x1t_ref[...]
    x2b = x2t_ref[...]
    wts = wts_ref[...]
    W1T = wts[:, 0:15]
    MAT = wts[:, 16:31]
    MBT = wts[:, 32:36]
    b1c = wts[:, 48:49]
    c2c = wts[:, 49:50]

    h0t = jnp.maximum(
        jnp.dot(W1T, x1b, preferred_element_type=_F32) + b1c, 0.0)
    c2t = (jnp.dot(MAT, x1b, preferred_element_type=_F32)
           + jnp.dot(MBT, x2b, preferred_element_type=_F32) + c2c)

    A = A_ref[...]
    h0n = jnp.transpose(h0t)
    c2n = jnp.transpose(c2t)
    h0_ref[...] = jnp.concatenate(
        [jnp.dot(A[r * KB:(r + 1) * KB], h0n, preferred_element_type=_F32)
         for r in range(8)], axis=1)
    c2_ref[...] = jnp.concatenate(
        [jnp.dot(A[r * KB:(r + 1) * KB], c2n, preferred_element_type=_F32)
         for r in range(8)], axis=1)

    lane = i * BC + lax.broadcasted_iota(jnp.int32, (16, BC), 1)
    m = lane < N
    ps1 = jnp.sum(jnp.where(m[:15], x1b, 0.0), axis=1, keepdims=True)
    ps2 = jnp.sum(jnp.where(m[:4], x2b, 0.0), axis=1, keepdims=True)
    col0 = jnp.concatenate([ps1, jnp.zeros((1, 1), _F32)], axis=0)
    col2 = jnp.concatenate([ps2, jnp.zeros((12, 1), _F32)], axis=0)
    ps = jnp.concatenate(
        [col0, jnp.zeros((16, 1), _F32), col2, jnp.zeros((16, 125), _F32)],
        axis=1)

    @pl.when(i == 0)
    def _():
        psum_ref[...] = ps

    @pl.when(i > 0)
    def _():
        psum_ref[...] += ps


def _y0_body(p_ref, h0_ref, c2_ref, spread_ref, WB1_ref, dis_ref, y0_ref):
    cnt = (p_ref[0] + p_ref[1]).astype(_F32)
    deg = jnp.dot(cnt, spread_ref[...], preferred_element_type=_F32) + 1.0
    dis = lax.rsqrt(deg)
    dis_ref[...] = dis
    y0 = (jnp.dot(h0_ref[...], WB1_ref[...],
                  preferred_element_type=_F32) + c2_ref[...]) * dis
    y0_ref[...] = y0.astype(_BF16)


def _layer_body(s_ref, y_ref, dis_ref, c2_ref, WB1_ref, bgt_ref, ynew_ref):
    dis = dis_ref[...]
    s = (s_ref[0] + s_ref[1] + y_ref[...]).astype(_F32)
    h = jnp.maximum(dis * s + bgt_ref[...], 0.0)
    ynew = (jnp.dot(h, WB1_ref[...],
                    preferred_element_type=_F32) + c2_ref[...]) * dis
    ynew_ref[...] = ynew.astype(_BF16)


def _final_body(s_ref, y_ref, dis_ref, bgt_ref, out_ref):
    i = pl.program_id(0)
    s = (s_ref[0] + s_ref[1] + y_ref[...]).astype(_F32)
    h = jnp.maximum(dis_ref[...] * s + bgt_ref[...], 0.0)
    prid = i * BRP + lax.broadcasted_iota(jnp.int32, (BRP, 128), 0)
    ps = jnp.sum(jnp.where(prid < PK_VALID, h, 0.0), axis=0, keepdims=True)

    @pl.when(i == 0)
    def _():
        out_ref[...] = ps

    @pl.when(i > 0)
    def _():
        out_ref[...] += ps


def _const_spec(shape):
    nd = len(shape)
    return pl.BlockSpec(shape, lambda i: (0,) * nd)


_pre_call = pl.pallas_call(
    _pre_body,
    grid=(GPRE,),
    in_specs=[
        pl.BlockSpec((15, BC), lambda i: (0, i)),
        pl.BlockSpec((4, BC), lambda i: (0, i)),
        _const_spec((16, 128)),
        _const_spec((BC, BC)),
    ],
    out_specs=[
        pl.BlockSpec((KB, 128), lambda i: (i, 0)),
        pl.BlockSpec((KB, 128), lambda i: (i, 0)),
        pl.BlockSpec((16, 128), lambda i: (0, 0)),
    ],
    out_shape=[
        jax.ShapeDtypeStruct((PK, 128), _F32),
        jax.ShapeDtypeStruct((PK, 128), _F32),
        jax.ShapeDtypeStruct((16, 128), _F32),
    ],
)

_y0_call = pl.pallas_call(
    _y0_body,
    grid=(GPK,),
    in_specs=[
        pl.BlockSpec((2, BRP, 128), lambda i: (0, i, 0)),
        pl.BlockSpec((BRP, 128), lambda i: (i, 0)),
        pl.BlockSpec((BRP, 128), lambda i: (i, 0)),
        _const_spec((128, 128)),
        _const_spec((128, 128)),
    ],
    out_specs=[
        pl.BlockSpec((BRP, 128), lambda i: (i, 0)),
        pl.BlockSpec((BRP, 128), lambda i: (i, 0)),
    ],
    out_shape=[
        jax.ShapeDtypeStruct((PK, 128), _F32),
        jax.ShapeDtypeStruct((PK, 128), _BF16),
    ],
)

_layer_call = pl.pallas_call(
    _layer_body,
    grid=(GPK,),
    in_specs=[
        pl.BlockSpec((2, BRP, 128), lambda i: (0, i, 0)),
        pl.BlockSpec((BRP, 128), lambda i: (i, 0)),
        pl.BlockSpec((BRP, 128), lambda i: (i, 0)),
        pl.BlockSpec((BRP, 128), lambda i: (i, 0)),
        _const_spec((128, 128)),
        _const_spec((1, 128)),
    ],
    out_specs=pl.BlockSpec((BRP, 128), lambda i: (i, 0)),
    out_shape=jax.ShapeDtypeStruct((PK, 128), _BF16),
)

_final_call = pl.pallas_call(
    _final_body,
    grid=(GPK,),
    in_specs=[
        pl.BlockSpec((2, BRP, 128), lambda i: (0, i, 0)),
        pl.BlockSpec((BRP, 128), lambda i: (i, 0)),
        pl.BlockSpec((BRP, 128), lambda i: (i, 0)),
        _const_spec((1, 128)),
    ],
    out_specs=pl.BlockSpec((1, 128), lambda i: (0, 0)),
    out_shape=jax.ShapeDtypeStruct((1, 128), _F32),
)

_PERM = np.zeros((BC, BC), np.float32)
for _q in range(BC):
    _PERM[_q, 8 * (_q % KB) + (_q // KB)] = 1.0


def kernel(x1, x2, edges, W1, b1, Wg, bg, W3, b3, W4, b4):
    x1t = jnp.transpose(x1)
    x2t = jnp.tile(jnp.transpose(x2), (1, 20))
    edges_pk = edges.reshape(2 * NCHT, 128)

    W4a = jnp.zeros((16, 32), _F32).at[:15, :19].set(W4[:15])
    W4b16 = jnp.zeros((16, 32), _F32).at[:4, :19].set(W4[15:])
    Wg2 = jnp.zeros((32, 16), _F32).at[:19, :15].set(Wg[15:])
    wts = jnp.zeros((16, 128), _F32)
    wts = wts.at[:15, 0:15].set(W1.T)
    wts = wts.at[:, 16:31].set((W4a @ Wg2).T[:, :15])
    wts = wts.at[:, 32:36].set((W4b16 @ Wg2).T[:, :4])
    wts = wts.at[:15, 48].set(b1)
    wts = wts.at[:15, 49].set(b4 @ Wg[15:])
    A = jnp.asarray(_PERM)

    eye8 = jnp.eye(8, dtype=_F32)
    Wg1 = jnp.zeros((16, 16), _F32).at[:15, :15].set(Wg[:15])
    WB1 = jnp.kron(eye8, Wg1)
    spread = jnp.kron(eye8, jnp.zeros((16, 16), _F32).at[0, :].set(1.0))
    bgt = jnp.tile(jnp.zeros((16,), _F32).at[:15].set(bg), 8)[None, :]

    zeros_np16 = jnp.zeros((NP, 16), _BF16)
    ones_e0 = jnp.zeros((PK, 128), _BF16).at[:, 0::16].set(1.0)

    h0_pk, c2_pk, psums = _pre_call(x1t, x2t, wts, A)

    def step(carry, it):
        y, dis, _, _ = carry
        s = _sc_scatter(y.reshape(NP, 16), edges_pk, zeros_np16)
        s_pk = s.reshape(NCORE, PK, 128)

        def b0(_):
            d, y0 = _y0_call(s_pk, h0_pk, c2_pk, spread, WB1)
            return d, y0

        def bn(_):
            return dis, _layer_call(s_pk, y, dis, c2_pk, WB1, bgt)

        dis2, ynew = lax.cond(it == 0, b0, bn, None)
        return (ynew, dis2, s_pk, y), None

    carry0 = (ones_e0,
              jnp.zeros((PK, 128), _F32),
              jnp.zeros((NCORE, PK, 128), _BF16),
              jnp.zeros((PK, 128), _BF16))
    (_, dis, s5_pk, y4), _ = lax.scan(step, carry0, jnp.arange(6))
    ch5 = _final_call(s5_pk, y4, dis, bgt)

    ch5_16 = ch5.reshape(8, 16).sum(axis=0)
    x1sum = psums[:15, 0]
    x2sum = psums[:4, 2]
    bsum = x1sum @ W4[:15] + x2sum @ W4[15:] + N * b4
    val = (ch5_16[:15] @ W3[:15, 0] + bsum @ W3[15:, 0]) / N + b3[0]
    return jnp.tanh(val)


# speedup vs baseline: 50.5532x; 1.1053x over previous
import functools

import jax
import jax.numpy as jnp
import numpy as np
from jax import lax
from jax.experimental import pallas as pl
from jax.experimental.pallas import tpu as pltpu
from jax.experimental.pallas import tpu_sc as plsc

N = 100000
NP = N + 96
PK = NP // 8
PK_VALID = N // 8
NW = 32
NCORE = 2
NSUB = 16
ZR = NP // NSUB

E = 1600000
NCHT = E // 128
CPW = NCHT // NW
XTRA = NCHT - CPW * NW
SP = 98

NBUF = 12
PF = 6
BRP = 368
GPK = PK // BRP

BC = 1280
KB = BC // 8
GPRE = -(-N // BC)

_F32 = jnp.float32
_BF16 = jnp.bfloat16


def _make_sc_scatter():
    mesh = plsc.VectorSubcoreMesh(
        core_axis_name="c", subcore_axis_name="s",
        num_cores=NCORE, num_subcores=NSUB)

    @functools.partial(
        pl.kernel,
        out_type=jax.ShapeDtypeStruct((NCORE, NP, 16), _BF16),
        mesh=mesh,
        scratch_types=[
            pltpu.VMEM((SP, 128), jnp.int32),
            pltpu.VMEM((SP, 128), jnp.int32),
            pltpu.VMEM((NBUF, 128, 16), _BF16),
            pltpu.VMEM_SHARED((NP, 16), _BF16),
            pltpu.SemaphoreType.DMA((NBUF,)),
            pltpu.SemaphoreType.DMA,
        ],
        compiler_params=pltpu.CompilerParams(use_tc_tiling_on_sc=False),
    )
    def sc_scatter(y_hbm, edges_hbm, zeros_hbm, out_hbm,
                   ridx, cidx, gbuf, acc, gsem, ssem):
        c = lax.axis_index("c")
        s = lax.axis_index("s")
        w = c * NSUB + s
        pltpu.sync_copy(zeros_hbm.at[pl.ds(s * ZR, ZR)],
                        acc.at[pl.ds(s * ZR, ZR)])
        plsc.subcore_barrier()

        def gstart(k):
            pltpu.async_copy(y_hbm.at[ridx.at[k]], gbuf.at[k % NBUF],
                             gsem.at[k % NBUF])

        def gwait(j):
            pltpu.make_async_copy(y_hbm.at[ridx.at[j]], gbuf.at[j % NBUF],
                                  gsem.at[j % NBUF]).wait()

        def sdrain():
            pltpu.make_async_copy(y_hbm.at[pl.ds(0, 128)], gbuf.at[0],
                                  ssem).wait()

        for p in range(4):
            lo = p * SP
            cnt = min(SP, CPW - lo)
            pltpu.sync_copy(edges_hbm.at[pl.ds(w * CPW + lo, cnt)],
                            ridx.at[pl.ds(0, cnt)])
            pltpu.sync_copy(edges_hbm.at[pl.ds(NCHT + w * CPW + lo, cnt)],
                            cidx.at[pl.ds(0, cnt)])
            if p == 3:
                @pl.when(w < XTRA)
                def _():
                    pltpu.sync_copy(edges_hbm.at[pl.ds(NW * CPW + w, 1)],
                                    ridx.at[pl.ds(cnt, 1)])
                    pltpu.sync_copy(
                        edges_hbm.at[pl.ds(NCHT + NW * CPW + w, 1)],
                        cidx.at[pl.ds(cnt, 1)])
                nloc = cnt + jnp.where(w < XTRA, 1, 0)
            else:
                nloc = cnt

            for k in range(PF):
                @pl.when(k < nloc)
                def _(k=k):
                    gstart(k)

            def body(j, carry2):
                gwait(j)
                pltpu.async_copy(gbuf.at[j % NBUF], acc.at[cidx.at[j]],
                                 ssem, add=True)

                @pl.when(j >= PF)
                def _():
                    sdrain()

                @pl.when(j + PF < nloc)
                def _():
                    gstart(j + PF)

                return carry2

            lax.fori_loop(0, nloc, body, 0)
            for k in range(PF):
                @pl.when(k < nloc)
                def _():
                    sdrain()

        plsc.subcore_barrier()
        pltpu.sync_copy(acc.at[pl.ds(s * ZR, ZR)],
                        out_hbm.at[c].at[pl.ds(s * ZR, ZR)])

    return sc_scatter


_SC_CACHE = {}


def _sc_scatter(*args):
    if "k" not in _SC_CACHE:
        _SC_CACHE["k"] = _make_sc_scatter()
    return _SC_CACHE["k"](*args)


def _pre_body(x1t_ref, x2t_ref, wts_ref, A_ref, h0_ref, c2_ref, psum_ref):
    i = pl.program_id(0)
    x1b = x1t_ref[...]
    x2b = x2t_ref[...]
    wts = wts_ref[...]
    W1T = wts[:, 0:15]
    MAT = wts[:, 16:31]
    MBT = wts[:, 32:36]
    b1c = wts[:, 48:49]
    c2c = wts[:, 49:50]

    h0t = jnp.maximum(
        jnp.dot(W1T, x1b, preferred_element_type=_F32) + b1c, 0.0)
    c2t = (jnp.dot(MAT, x1b, preferred_element_type=_F32)
           + jnp.dot(MBT, x2b, preferred_element_type=_F32) + c2c)

    A = A_ref[...]
    h0n = jnp.transpose(h0t)
    c2n = jnp.transpose(c2t)
    h0_ref[...] = jnp.concatenate(
        [jnp.dot(A[r * KB:(r + 1) * KB], h0n, preferred_element_type=_F32)
         for r in range(8)], axis=1).astype(_BF16)
    c2_ref[...] = jnp.concatenate(
        [jnp.dot(A[r * KB:(r + 1) * KB], c2n, preferred_element_type=_F32)
         for r in range(8)], axis=1).astype(_BF16)

    lane = i * BC + lax.broadcasted_iota(jnp.int32, (16, BC), 1)
    m = lane < N
    ps1 = jnp.sum(jnp.where(m[:15], x1b, 0.0), axis=1, keepdims=True)
    ps2 = jnp.sum(jnp.where(m[:4], x2b, 0.0), axis=1, keepdims=True)
    col0 = jnp.concatenate([ps1, jnp.zeros((1, 1), _F32)], axis=0)
    col2 = jnp.concatenate([ps2, jnp.zeros((12, 1), _F32)], axis=0)
    ps = jnp.concatenate(
        [col0, jnp.zeros((16, 1), _F32), col2, jnp.zeros((16, 125), _F32)],
        axis=1)

    @pl.when(i == 0)
    def _():
        psum_ref[...] = ps

    @pl.when(i > 0)
    def _():
        psum_ref[...] += ps


def _y0_body(p_ref, h0_ref, c2_ref, spread_ref, WB1_ref, dis_ref, y0_ref):
    cnt = (p_ref[0] + p_ref[1]).astype(_F32)
    deg = jnp.dot(cnt, spread_ref[...], preferred_element_type=_F32) + 1.0
    dis = lax.rsqrt(deg)
    dis_ref[...] = dis
    y0 = (jnp.dot(h0_ref[...].astype(_F32), WB1_ref[...],
                  preferred_element_type=_F32)
          + c2_ref[...].astype(_F32)) * dis
    y0_ref[...] = y0.astype(_BF16)


def _layer_body(s_ref, y_ref, dis_ref, c2_ref, WB1_ref, bgt_ref, ynew_ref):
    dis = dis_ref[...]
    s = (s_ref[0] + s_ref[1] + y_ref[...]).astype(_F32)
    h = jnp.maximum(dis * s + bgt_ref[...], 0.0)
    ynew = (jnp.dot(h, WB1_ref[...],
                    preferred_element_type=_F32)
            + c2_ref[...].astype(_F32)) * dis
    ynew_ref[...] = ynew.astype(_BF16)


def _final_body(s_ref, y_ref, dis_ref, bgt_ref, out_ref):
    i = pl.program_id(0)
    s = (s_ref[0] + s_ref[1] + y_ref[...]).astype(_F32)
    h = jnp.maximum(dis_ref[...] * s + bgt_ref[...], 0.0)
    prid = i * BRP + lax.broadcasted_iota(jnp.int32, (BRP, 128), 0)
    ps = jnp.sum(jnp.where(prid < PK_VALID, h, 0.0), axis=0, keepdims=True)

    @pl.when(i == 0)
    def _():
        out_ref[...] = ps

    @pl.when(i > 0)
    def _():
        out_ref[...] += ps


def _const_spec(shape):
    nd = len(shape)
    return pl.BlockSpec(shape, lambda i: (0,) * nd)


_pre_call = pl.pallas_call(
    _pre_body,
    grid=(GPRE,),
    in_specs=[
        pl.BlockSpec((15, BC), lambda i: (0, i)),
        pl.BlockSpec((4, BC), lambda i: (0, i)),
        _const_spec((16, 128)),
        _const_spec((BC, BC)),
    ],
    out_specs=[
        pl.BlockSpec((KB, 128), lambda i: (i, 0)),
        pl.BlockSpec((KB, 128), lambda i: (i, 0)),
        pl.BlockSpec((16, 128), lambda i: (0, 0)),
    ],
    out_shape=[
        jax.ShapeDtypeStruct((PK, 128), _BF16),
        jax.ShapeDtypeStruct((PK, 128), _BF16),
        jax.ShapeDtypeStruct((16, 128), _F32),
    ],
)

_y0_call = pl.pallas_call(
    _y0_body,
    grid=(GPK,),
    in_specs=[
        pl.BlockSpec((2, BRP, 128), lambda i: (0, i, 0)),
        pl.BlockSpec((BRP, 128), lambda i: (i, 0)),
        pl.BlockSpec((BRP, 128), lambda i: (i, 0)),
        _const_spec((128, 128)),
        _const_spec((128, 128)),
    ],
    out_specs=[
        pl.BlockSpec((BRP, 128), lambda i: (i, 0)),
        pl.BlockSpec((BRP, 128), lambda i: (i, 0)),
    ],
    out_shape=[
        jax.ShapeDtypeStruct((PK, 128), _F32),
        jax.ShapeDtypeStruct((PK, 128), _BF16),
    ],
)

_layer_call = pl.pallas_call(
    _layer_body,
    grid=(GPK,),
    in_specs=[
        pl.BlockSpec((2, BRP, 128), lambda i: (0, i, 0)),
        pl.BlockSpec((BRP, 128), lambda i: (i, 0)),
        pl.BlockSpec((BRP, 128), lambda i: (i, 0)),
        pl.BlockSpec((BRP, 128), lambda i: (i, 0)),
        _const_spec((128, 128)),
        _const_spec((1, 128)),
    ],
    out_specs=pl.BlockSpec((BRP, 128), lambda i: (i, 0)),
    out_shape=jax.ShapeDtypeStruct((PK, 128), _BF16),
)

_final_call = pl.pallas_call(
    _final_body,
    grid=(GPK,),
    in_specs=[
        pl.BlockSpec((2, BRP, 128), lambda i: (0, i, 0)),
        pl.BlockSpec((BRP, 128), lambda i: (i, 0)),
        pl.BlockSpec((BRP, 128), lambda i: (i, 0)),
        _const_spec((1, 128)),
    ],
    out_specs=pl.BlockSpec((1, 128), lambda i: (0, 0)),
    out_shape=jax.ShapeDtypeStruct((1, 128), _F32),
)

_PERM = np.zeros((BC, BC), np.float32)
for _q in range(BC):
    _PERM[_q, 8 * (_q % KB) + (_q // KB)] = 1.0


def kernel(x1, x2, edges, W1, b1, Wg, bg, W3, b3, W4, b4):
    x1t = jnp.transpose(x1)
    x2t = jnp.tile(jnp.transpose(x2), (1, 20))
    edges_pk = edges.reshape(2 * NCHT, 128)

    W4a = jnp.zeros((16, 32), _F32).at[:15, :19].set(W4[:15])
    W4b16 = jnp.zeros((16, 32), _F32).at[:4, :19].set(W4[15:])
    Wg2 = jnp.zeros((32, 16), _F32).at[:19, :15].set(Wg[15:])
    wts = jnp.zeros((16, 128), _F32)
    wts = wts.at[:15, 0:15].set(W1.T)
    wts = wts.at[:, 16:31].set((W4a @ Wg2).T[:, :15])
    wts = wts.at[:, 32:36].set((W4b16 @ Wg2).T[:, :4])
    wts = wts.at[:15, 48].set(b1)
    wts = wts.at[:15, 49].set(b4 @ Wg[15:])
    A = jnp.asarray(_PERM)

    eye8 = jnp.eye(8, dtype=_F32)
    Wg1 = jnp.zeros((16, 16), _F32).at[:15, :15].set(Wg[:15])
    WB1 = jnp.kron(eye8, Wg1)
    spread = jnp.kron(eye8, jnp.zeros((16, 16), _F32).at[0, :].set(1.0))
    bgt = jnp.tile(jnp.zeros((16,), _F32).at[:15].set(bg), 8)[None, :]

    zeros_np16 = jnp.zeros((NP, 16), _BF16)
    ones_e0 = jnp.zeros((PK, 128), _BF16).at[:, 0::16].set(1.0)

    h0_pk, c2_pk, psums = _pre_call(x1t, x2t, wts, A)

    def step(carry, it):
        y, dis, _, _ = carry
        s = _sc_scatter(y.reshape(NP, 16), edges_pk, zeros_np16)
        s_pk = s.reshape(NCORE, PK, 128)

        def b0(_):
            d, y0 = _y0_call(s_pk, h0_pk, c2_pk, spread, WB1)
            return d, y0

        def bn(_):
            return dis, _layer_call(s_pk, y, dis, c2_pk, WB1, bgt)

        dis2, ynew = lax.cond(it == 0, b0, bn, None)
        return (ynew, dis2, s_pk, y), None

    carry0 = (ones_e0,
              jnp.zeros((PK, 128), _F32),
              jnp.zeros((NCORE, PK, 128), _BF16),
              jnp.zeros((PK, 128), _BF16))
    (_, dis, s5_pk, y4), _ = lax.scan(step, carry0, jnp.arange(6))
    ch5 = _final_call(s5_pk, y4, dis, bgt)

    ch5_16 = ch5.reshape(8, 16).sum(axis=0)
    x1sum = psums[:15, 0]
    x2sum = psums[:4, 2]
    bsum = x1sum @ W4[:15] + x2sum @ W4[15:] + N * b4
    val = (ch5_16[:15] @ W3[:15, 0] + bsum @ W3[15:, 0]) / N + b3[0]
    return jnp.tanh(val)
